# baseline (speedup 1.0000x reference)
import jax
import jax.numpy as jnp
from jax.experimental import pallas as pl
from jax.experimental.pallas import tpu as pltpu

_CHUNKS = (1024, 768, 640, 512, 384, 320, 256, 128, 64)


def _fused_kernel(s_hbm, e_hbm, c_hbm, out_ref,
                  s_v, e_v, c_v, sem_e, sem_s, sem_c):
    cp_s = pltpu.make_async_copy(s_hbm, s_v, sem_s)
    cp_s.start()
    cp_c = pltpu.make_async_copy(c_hbm, c_v, sem_c)
    cp_c.start()

    offs = []
    off = 0
    copies_e = []
    for i, ch in enumerate(_CHUNKS):
        cp = pltpu.make_async_copy(
            e_hbm.at[pl.ds(off, ch), :],
            e_v.at[pl.ds(off, ch), :],
            sem_e.at[i],
        )
        cp.start()
        copies_e.append(cp)
        offs.append(off)
        off += ch

    cp_c.wait()
    c_bf = c_v[...].astype(jnp.bfloat16)
    cp_s.wait()
    s_bf = s_v[...].astype(jnp.bfloat16)

    t = None
    for i, ch in enumerate(_CHUNKS):
        copies_e[i].wait()
        part = jax.lax.dot_general(
            c_bf[:, offs[i]:offs[i] + ch],
            e_v[offs[i]:offs[i] + ch, :].astype(jnp.bfloat16),
            (((1,), (0,)), ((), ())),
            preferred_element_type=jnp.float32,
        )
        t = part if t is None else t + part

    out_ref[...] = jax.lax.dot_general(
        s_bf, t.astype(jnp.bfloat16),
        (((1,), (1,)), ((), ())),
        preferred_element_type=jnp.float32,
    )


def kernel(samples, enc_weight, cent_weight):
    batch, n_features = samples.shape
    n_classes, n_dims = cent_weight.shape
    assert sum(_CHUNKS) == n_dims
    return pl.pallas_call(
        _fused_kernel,
        in_specs=[
            pl.BlockSpec(memory_space=pl.ANY),
            pl.BlockSpec(memory_space=pl.ANY),
            pl.BlockSpec(memory_space=pl.ANY),
        ],
        out_specs=pl.BlockSpec(memory_space=pltpu.VMEM),
        out_shape=jax.ShapeDtypeStruct((batch, n_classes), jnp.float32),
        scratch_shapes=[
            pltpu.VMEM((batch, n_features), jnp.float32),
            pltpu.VMEM((n_dims, n_features), jnp.float32),
            pltpu.VMEM((n_classes, n_dims), jnp.float32),
            pltpu.SemaphoreType.DMA((len(_CHUNKS),)),
            pltpu.SemaphoreType.DMA,
            pltpu.SemaphoreType.DMA,
        ],
    )(samples, enc_weight, cent_weight)


# speedup vs baseline: 1.1055x; 1.1055x over previous
import jax
import jax.numpy as jnp
from jax.experimental import pallas as pl
from jax.experimental.pallas import tpu as pltpu

_CHUNKS = (128, 256, 256, 384, 512, 640, 896, 1024)
_NS = 2


def _fused_kernel(s_hbm, e_hbm, c_hbm, out_ref,
                  s_v, e_v, c_v, sem_e, sem_s, sem_c):
    n = len(_CHUNKS)
    offs = []
    off = 0
    copies_c, copies_e = [], []
    for i, ch in enumerate(_CHUNKS):
        cp = pltpu.make_async_copy(
            c_hbm.at[:, pl.ds(off, ch)],
            c_v.at[:, pl.ds(off, ch)],
            sem_c.at[i],
        )
        cp.start()
        copies_c.append(cp)
        offs.append(off)
        off += ch
    for i, ch in enumerate(_CHUNKS):
        cp = pltpu.make_async_copy(
            e_hbm.at[pl.ds(offs[i], ch), :],
            e_v.at[pl.ds(offs[i], ch), :],
            sem_e.at[i],
        )
        cp.start()
        copies_e.append(cp)
    b_total = s_hbm.shape[0]
    bs = b_total // _NS
    copies_s = []
    for i in range(_NS):
        cp = pltpu.make_async_copy(
            s_hbm.at[pl.ds(i * bs, bs), :],
            s_v.at[pl.ds(i * bs, bs), :],
            sem_s.at[i],
        )
        cp.start()
        copies_s.append(cp)

    t = None
    for i, ch in enumerate(_CHUNKS):
        copies_c[i].wait()
        copies_e[i].wait()
        part = jax.lax.dot_general(
            c_v[:, offs[i]:offs[i] + ch],
            e_v[offs[i]:offs[i] + ch, :],
            (((1,), (0,)), ((), ())),
            preferred_element_type=jnp.float32,
        )
        t = part if t is None else t + part

    for i in range(_NS):
        copies_s[i].wait()
        out_ref[i * bs:(i + 1) * bs, :] = jax.lax.dot_general(
            s_v[i * bs:(i + 1) * bs, :], t,
            (((1,), (1,)), ((), ())),
            preferred_element_type=jnp.float32,
        )


def kernel(samples, enc_weight, cent_weight):
    batch, n_features = samples.shape
    n_classes, n_dims = cent_weight.shape
    assert sum(_CHUNKS) == n_dims
    return pl.pallas_call(
        _fused_kernel,
        in_specs=[
            pl.BlockSpec(memory_space=pl.ANY),
            pl.BlockSpec(memory_space=pl.ANY),
            pl.BlockSpec(memory_space=pl.ANY),
        ],
        out_specs=pl.BlockSpec(memory_space=pltpu.VMEM),
        out_shape=jax.ShapeDtypeStruct((batch, n_classes), jnp.float32),
        scratch_shapes=[
            pltpu.VMEM((batch, n_features), jnp.float32),
            pltpu.VMEM((n_dims, n_features), jnp.float32),
            pltpu.VMEM((n_classes, n_dims), jnp.float32),
            pltpu.SemaphoreType.DMA((len(_CHUNKS),)),
            pltpu.SemaphoreType.DMA((_NS,)),
            pltpu.SemaphoreType.DMA((len(_CHUNKS),)),
        ],
    )(samples, enc_weight, cent_weight)


# speedup vs baseline: 1.1771x; 1.0648x over previous
import jax
import jax.numpy as jnp
from jax.experimental import pallas as pl
from jax.experimental.pallas import tpu as pltpu

_CHUNKS = (128, 256, 256, 384, 512, 640, 896, 1024)


def _fused_kernel(s_hbm, e_hbm, c_hbm, out_ref,
                  s_v, e_v, c_v, sem_e, sem_s, sem_c):
    cp_c = pltpu.make_async_copy(c_hbm, c_v, sem_c)
    cp_c.start()
    offs = []
    off = 0
    copies_e = []
    for i, ch in enumerate(_CHUNKS):
        cp = pltpu.make_async_copy(
            e_hbm.at[pl.ds(off, ch), :],
            e_v.at[pl.ds(off, ch), :],
            sem_e.at[i],
        )
        cp.start()
        copies_e.append(cp)
        offs.append(off)
        off += ch
    cp_s = pltpu.make_async_copy(s_hbm, s_v, sem_s)
    cp_s.start()

    cp_c.wait()
    t = None
    for i, ch in enumerate(_CHUNKS):
        copies_e[i].wait()
        part = jax.lax.dot_general(
            c_v[:, offs[i]:offs[i] + ch],
            e_v[offs[i]:offs[i] + ch, :],
            (((1,), (0,)), ((), ())),
            preferred_element_type=jnp.float32,
        )
        t = part if t is None else t + part

    cp_s.wait()
    out_ref[...] = jax.lax.dot_general(
        s_v[...], t,
        (((1,), (1,)), ((), ())),
        preferred_element_type=jnp.float32,
    )


def kernel(samples, enc_weight, cent_weight):
    batch, n_features = samples.shape
    n_classes, n_dims = cent_weight.shape
    assert sum(_CHUNKS) == n_dims
    return pl.pallas_call(
        _fused_kernel,
        in_specs=[
            pl.BlockSpec(memory_space=pl.ANY),
            pl.BlockSpec(memory_space=pl.ANY),
            pl.BlockSpec(memory_space=pl.ANY),
        ],
        out_specs=pl.BlockSpec(memory_space=pltpu.VMEM),
        out_shape=jax.ShapeDtypeStruct((batch, n_classes), jnp.float32),
        scratch_shapes=[
            pltpu.VMEM((batch, n_features), jnp.float32),
            pltpu.VMEM((n_dims, n_features), jnp.float32),
            pltpu.VMEM((n_classes, n_dims), jnp.float32),
            pltpu.SemaphoreType.DMA((len(_CHUNKS),)),
            pltpu.SemaphoreType.DMA,
            pltpu.SemaphoreType.DMA,
        ],
    )(samples, enc_weight, cent_weight)
